# baseline (speedup 1.0000x reference)
import jax
import jax.numpy as jnp
from jax import lax
from jax.experimental import pallas as pl
from jax.experimental.pallas import tpu as pltpu
from jax.experimental.pallas import tpu_sc as plsc

B = 16384
D = 128
NC = 2
NS = 16
NW = NC * NS
RPW = B // NW
NG = RPW // 16
NT = 2400


def _fold_body(at_ref, gt_ref, vtt_ref, wc_ref, bc_ref, wo_ref, bo_ref,
               t_ref, wc2_ref):
    w = wo_ref[...]
    bias = bc_ref[...] @ w[96:128, :] + bo_ref[...]
    a2 = at_ref[...] @ w[0:32, :] + bias[None, :]
    g2 = gt_ref[...] @ w[32:64, :]
    v2 = lax.dot_general(vtt_ref[...], w[64:96, :],
                         (((0,), (0,)), ((), ())))
    wc2_ref[...] = (wc_ref[...] @ w[96:128, :]).reshape(-1)

    def onehot(vals, n):
        k = lax.broadcasted_iota(jnp.int32, (NT, n), 1)
        return (vals[:, None] == k).astype(jnp.float32)

    r = lax.broadcasted_iota(jnp.int32, (NT,), 0)
    t = onehot(r // 600, 4) @ a2
    t = t + onehot((r // 60) % 10, 10) @ g2
    t = t + onehot(r % 60, 60) @ v2
    t_ref[...] = t


def _fold_tables(asset_table, geo_table, vin_table, W_cont, b_cont, W_out,
                 b_out):
    return pl.pallas_call(
        _fold_body,
        out_shape=(
            jax.ShapeDtypeStruct((NT, D), jnp.float32),
            jax.ShapeDtypeStruct((4 * D,), jnp.float32),
        ),
    )(asset_table, geo_table, vin_table.T, W_cont, b_cont, W_out, b_out)


def _sc_body(a_hbm, g_hbm, v_hbm, cf_hbm, t_hbm, wc2_hbm, out_hbm,
             a_v, g_v, v_v, cf_v, wc2_v, idx_v, out_v,
             sem_in, sem_cf, sem_g, sem_out):
    wid = lax.axis_index("s") * NC + lax.axis_index("c")
    base = wid * RPW

    c_a = pltpu.async_copy(a_hbm.at[pl.ds(base, RPW)], a_v, sem_in)
    c_g = pltpu.async_copy(g_hbm.at[pl.ds(base, RPW)], g_v, sem_in)
    c_v = pltpu.async_copy(v_hbm.at[pl.ds(base, RPW)], v_v, sem_in)
    c_cf = pltpu.async_copy(cf_hbm.at[:, pl.ds(base, RPW)], cf_v, sem_cf)
    c_w = pltpu.async_copy(wc2_hbm, wc2_v, sem_cf)
    c_a.wait()
    c_g.wait()
    c_v.wait()

    gathers = []
    for k in range(4):
        for q in range(8):
            gi = k * 8 + q
            i16 = (a_v[pl.ds(gi * 16, 16)] * 600
                   + g_v[pl.ds(gi * 16, 16)] * 60
                   + v_v[pl.ds(gi * 16, 16)])
            idx_v[k, pl.ds(q * 16, 16)] = i16
        gathers.append(pltpu.async_copy(
            t_hbm.at[idx_v.at[k]], out_v.at[pl.ds(k * 128, 128)], sem_g))

    c_cf.wait()
    c_w.wait()
    wc2 = [[wc2_v[pl.ds(j * D + c * 16, 16)] for c in range(8)]
           for j in range(4)]

    def group(gi):
        row0 = gi * 16
        cfq = [cf_v[j, pl.ds(row0, 16)] for j in range(4)]
        for c in range(8):
            off = c * 16
            w = [wc2[j][c] for j in range(4)]
            for l in range(16):
                cj = [cfq[j][l] for j in range(4)]
                upd = (cj[0] * w[0] + cj[1] * w[1]) + \
                      (cj[2] * w[2] + cj[3] * w[3])
                out_v[row0 + l, pl.ds(off, 16)] = \
                    out_v[row0 + l, pl.ds(off, 16)] + upd

    outs = []
    for k in range(4):
        gathers[k].wait()
        plsc.parallel_loop(k * 8, (k + 1) * 8, 1)(group)
        outs.append(pltpu.async_copy(
            out_v.at[pl.ds(k * 128, 128)],
            out_hbm.at[pl.ds(base + k * 128, 128)], sem_out))
    for k in range(4):
        outs[k].wait()


def kernel(asset_class, geography, vintage, continuous_features,
           asset_table, geo_table, vin_table, W_cont, b_cont, W_out, b_out):
    t, wc2 = _fold_tables(
        asset_table, geo_table, vin_table, W_cont, b_cont, W_out, b_out)

    sc = pl.kernel(
        _sc_body,
        out_type=jax.ShapeDtypeStruct((B, D), jnp.float32),
        mesh=plsc.VectorSubcoreMesh(core_axis_name="c", subcore_axis_name="s"),
        scratch_types=[
            pltpu.VMEM((RPW,), jnp.int32),
            pltpu.VMEM((RPW,), jnp.int32),
            pltpu.VMEM((RPW,), jnp.int32),
            pltpu.VMEM((4, RPW), jnp.float32),
            pltpu.VMEM((4 * D,), jnp.float32),
            pltpu.VMEM((4, 128), jnp.int32),
            pltpu.VMEM((RPW, D), jnp.float32),
            pltpu.SemaphoreType.DMA,
            pltpu.SemaphoreType.DMA,
            pltpu.SemaphoreType.DMA,
            pltpu.SemaphoreType.DMA,
        ],
    )
    return sc(asset_class.astype(jnp.int32), geography.astype(jnp.int32),
              vintage.astype(jnp.int32), continuous_features.T, t, wc2)


# speedup vs baseline: 1.0925x; 1.0925x over previous
import jax
import jax.numpy as jnp
from jax import lax
from jax.experimental import pallas as pl
from jax.experimental.pallas import tpu as pltpu
from jax.experimental.pallas import tpu_sc as plsc

B = 16384
D = 128
NC = 2
NS = 16
NW = NC * NS
RPW = B // NW
NG = RPW // 16
NT = 2400


def _fold_body(at_ref, gt_ref, vtt_ref, wc_ref, bc_ref, wo_ref, bo_ref,
               ac_ref, geo_ref, vin_ref, t_ref, wc2_ref, idx_ref):
    idx_ref[...] = ac_ref[...] * 600 + geo_ref[...] * 60 + vin_ref[...]
    w = wo_ref[...]
    bias = bc_ref[...] @ w[96:128, :] + bo_ref[...]
    a2 = at_ref[...] @ w[0:32, :] + bias[None, :]
    g2 = gt_ref[...] @ w[32:64, :]
    v2 = lax.dot_general(vtt_ref[...], w[64:96, :],
                         (((0,), (0,)), ((), ())))
    wc2_ref[...] = (wc_ref[...] @ w[96:128, :]).reshape(-1)

    def onehot(vals, n):
        k = lax.broadcasted_iota(jnp.int32, (NT, n), 1)
        return (vals[:, None] == k).astype(jnp.float32)

    r = lax.broadcasted_iota(jnp.int32, (NT,), 0)
    t = onehot(r // 600, 4) @ a2
    t = t + onehot((r // 60) % 10, 10) @ g2
    t = t + onehot(r % 60, 60) @ v2
    t_ref[...] = t


def _fold_tables(asset_table, geo_table, vin_table, W_cont, b_cont, W_out,
                 b_out, ac, geo, vin):
    return pl.pallas_call(
        _fold_body,
        out_shape=(
            jax.ShapeDtypeStruct((NT, D), jnp.float32),
            jax.ShapeDtypeStruct((4 * D,), jnp.float32),
            jax.ShapeDtypeStruct((B,), jnp.int32),
        ),
    )(asset_table, geo_table, vin_table.T, W_cont, b_cont, W_out, b_out,
      ac, geo, vin)


def _sc_body(idx_hbm, cf_hbm, t_hbm, wc2_hbm, out_hbm,
             cf_v, wc2_v, idx_v, out_v,
             sem_in, sem_cf, sem_g, sem_out):
    wid = lax.axis_index("s") * NC + lax.axis_index("c")
    base = wid * RPW

    idx_copies = [
        pltpu.async_copy(idx_hbm.at[pl.ds(base + k * 128, 128)],
                         idx_v.at[k], sem_in)
        for k in range(4)
    ]
    c_cf = pltpu.async_copy(cf_hbm.at[:, pl.ds(base, RPW)], cf_v, sem_cf)
    c_w = pltpu.async_copy(wc2_hbm, wc2_v, sem_cf)

    gathers = []
    for k in range(4):
        idx_copies[k].wait()
        gathers.append(pltpu.async_copy(
            t_hbm.at[idx_v.at[k]], out_v.at[pl.ds(k * 128, 128)], sem_g))

    c_cf.wait()
    c_w.wait()
    wc2 = [[wc2_v[pl.ds(j * D + c * 16, 16)] for c in range(8)]
           for j in range(4)]

    def group(gi):
        cfq = [cf_v[j, pl.ds(gi * 16, 16)] for j in range(4)]
        row0 = gi * 16
        for l in range(16):
            cj = [cfq[j][l] for j in range(4)]
            for c in range(8):
                off = c * 16
                upd = (cj[0] * wc2[0][c] + cj[1] * wc2[1][c]) + \
                      (cj[2] * wc2[2][c] + cj[3] * wc2[3][c])
                out_v[row0 + l, pl.ds(off, 16)] = \
                    out_v[row0 + l, pl.ds(off, 16)] + upd

    outs = []
    for k in range(4):
        gathers[k].wait()
        plsc.parallel_loop(k * 8, (k + 1) * 8, 1)(group)
        outs.append(pltpu.async_copy(
            out_v.at[pl.ds(k * 128, 128)],
            out_hbm.at[pl.ds(base + k * 128, 128)], sem_out))
    for k in range(4):
        outs[k].wait()


def kernel(asset_class, geography, vintage, continuous_features,
           asset_table, geo_table, vin_table, W_cont, b_cont, W_out, b_out):
    t, wc2, idx = _fold_tables(
        asset_table, geo_table, vin_table, W_cont, b_cont, W_out, b_out,
        asset_class.astype(jnp.int32), geography.astype(jnp.int32),
        vintage.astype(jnp.int32))

    sc = pl.kernel(
        _sc_body,
        out_type=jax.ShapeDtypeStruct((B, D), jnp.float32),
        mesh=plsc.VectorSubcoreMesh(core_axis_name="c", subcore_axis_name="s"),
        scratch_types=[
            pltpu.VMEM((4, RPW), jnp.float32),
            pltpu.VMEM((4 * D,), jnp.float32),
            pltpu.VMEM((4, 128), jnp.int32),
            pltpu.VMEM((RPW, D), jnp.float32),
            pltpu.SemaphoreType.DMA,
            pltpu.SemaphoreType.DMA,
            pltpu.SemaphoreType.DMA,
            pltpu.SemaphoreType.DMA,
        ],
    )
    return sc(idx, continuous_features.T, t, wc2)
